# baseline (speedup 1.0000x reference)
import functools

import jax
import jax.numpy as jnp
from jax import lax
from jax.experimental import pallas as pl
from jax.experimental.pallas import tpu as pltpu
from jax.experimental.pallas import tpu_sc as plsc

N = 10000
E = 320000
D = 128
K = 16
KD = K * D
ACC_W = 128
NC, NS = 2, 16
NW = NC * NS
EPW = E // NW
C = 16
NCHUNK = EPW // C
NP = 10240
ROWS_PER_TILE = NP // NS
ZROWS = 32
EPS = 1e-5


def _mm_pack_body(x_ref, wlo_ref, whi_ref, o_ref):
    lo = jnp.dot(x_ref[...], wlo_ref[...], preferred_element_type=jnp.float32)
    hi = jnp.dot(x_ref[...], whi_ref[...], preferred_element_type=jnp.float32)
    lou = lax.bitcast_convert_type(lo.astype(jnp.bfloat16),
                                   jnp.uint16).astype(jnp.uint32)
    hiu = lax.bitcast_convert_type(hi.astype(jnp.bfloat16),
                                   jnp.uint16).astype(jnp.uint32)
    o_ref[...] = lax.bitcast_convert_type(lou | (hiu << 16), jnp.int32)


def _mm_pack(x, wlo, whi):
    n, d = x.shape
    p = wlo.shape[1]
    blk = 1000
    return pl.pallas_call(
        _mm_pack_body,
        grid=(n // blk,),
        in_specs=[pl.BlockSpec((blk, d), lambda i: (i, 0)),
                  pl.BlockSpec((d, p), lambda i: (0, 0)),
                  pl.BlockSpec((d, p), lambda i: (0, 0))],
        out_specs=pl.BlockSpec((blk, p), lambda i: (i, 0)),
        out_shape=jax.ShapeDtypeStruct((n, p), jnp.int32),
    )(x, wlo, whi)


def _basis_body(a_ref, o_ref):
    a = a_ref[...]
    kk = lax.broadcasted_iota(jnp.int32, (a.shape[0], K), 1)
    w = jnp.ones((a.shape[0], K), jnp.float32)
    for d in range(4):
        ad = a[:, d:d + 1]
        w = w * jnp.where((kk >> d) & 1 == 1, ad, 1.0 - ad)
    o_ref[...] = w


def _basis(edge_attr):
    blk = 4000
    return pl.pallas_call(
        _basis_body,
        grid=(E // blk,),
        in_specs=[pl.BlockSpec((blk, 4), lambda i: (i, 0))],
        out_specs=pl.BlockSpec((blk, K), lambda i: (i, 0)),
        out_shape=jax.ShapeDtypeStruct((E, K), jnp.float32),
    )(edge_attr)


def _conv1_body(p_ref, c_ref, x_ref, r_ref, g_ref, b_ref, o_ref,
                hbuf, s_ref, q_ref):
    ph = pl.program_id(0)
    i = pl.program_id(1)

    @pl.when(ph == 0)
    def _():
        p = p_ref[0] + p_ref[1]
        cnt = c_ref[0, :, 0:1] + c_ref[1, :, 0:1]
        agg = p / jnp.maximum(cnt, 1.0)
        h = agg + jnp.dot(x_ref[...], r_ref[...],
                          preferred_element_type=jnp.float32)
        hbuf[pl.ds(i * 1000, 1000), :] = h

        @pl.when(i == 0)
        def _():
            s_ref[...] = jnp.zeros_like(s_ref)
            q_ref[...] = jnp.zeros_like(q_ref)

        s_ref[...] += jnp.sum(h, axis=0, keepdims=True)
        q_ref[...] += jnp.sum(h * h, axis=0, keepdims=True)

    @pl.when(ph == 1)
    def _():
        h = hbuf[pl.ds(i * 1000, 1000), :]
        o_ref[...] = _elu(_norm(h, s_ref[...], q_ref[...],
                                g_ref[...], b_ref[...]))


def _conv1_post(p, cnt, x, root, g, b):
    blk = 1000
    vec = pl.BlockSpec((1, D), lambda ph, i: (0, 0))
    return pl.pallas_call(
        _conv1_body,
        grid=(2, N // blk),
        in_specs=[pl.BlockSpec((NC, blk, ACC_W), lambda ph, i: (0, i, 0)),
                  pl.BlockSpec((NC, blk, ACC_W), lambda ph, i: (0, i, 0)),
                  pl.BlockSpec((blk, D), lambda ph, i: (i, 0)),
                  pl.BlockSpec((D, D), lambda ph, i: (0, 0)),
                  vec, vec],
        out_specs=pl.BlockSpec((blk, D), lambda ph, i: (i, 0)),
        out_shape=jax.ShapeDtypeStruct((N, D), jnp.float32),
        scratch_shapes=[pltpu.VMEM((N, D), jnp.float32),
                        pltpu.VMEM((1, D), jnp.float32),
                        pltpu.VMEM((1, D), jnp.float32)],
    )(p, cnt, x, root, g.reshape(1, D), b.reshape(1, D))


def _conv2_body(p_ref, c_ref, x_ref, r_ref, g2_ref, b2_ref,
                xs_ref, wl_ref, gs_ref, bs_ref, o_ref,
                hbuf, kbuf, s_ref, q_ref, ss_ref, qs_ref):
    ph = pl.program_id(0)
    i = pl.program_id(1)

    @pl.when(ph == 0)
    def _():
        p = p_ref[0] + p_ref[1]
        cnt = c_ref[0, :, 0:1] + c_ref[1, :, 0:1]
        agg = p / jnp.maximum(cnt, 1.0)
        h = agg + jnp.dot(x_ref[...], r_ref[...],
                          preferred_element_type=jnp.float32)
        hbuf[pl.ds(i * 1000, 1000), :] = h
        sk = jnp.dot(xs_ref[...], wl_ref[...],
                     preferred_element_type=jnp.float32)
        kbuf[pl.ds(i * 1000, 1000), :] = sk

        @pl.when(i == 0)
        def _():
            s_ref[...] = jnp.zeros_like(s_ref)
            q_ref[...] = jnp.zeros_like(q_ref)
            ss_ref[...] = jnp.zeros_like(ss_ref)
            qs_ref[...] = jnp.zeros_like(qs_ref)

        s_ref[...] += jnp.sum(h, axis=0, keepdims=True)
        q_ref[...] += jnp.sum(h * h, axis=0, keepdims=True)
        ss_ref[...] += jnp.sum(sk, axis=0, keepdims=True)
        qs_ref[...] += jnp.sum(sk * sk, axis=0, keepdims=True)

    @pl.when(ph == 1)
    def _():
        h = hbuf[pl.ds(i * 1000, 1000), :]
        sk = kbuf[pl.ds(i * 1000, 1000), :]
        hn = _norm(h, s_ref[...], q_ref[...], g2_ref[...], b2_ref[...])
        kn = _norm(sk, ss_ref[...], qs_ref[...], gs_ref[...], bs_ref[...])
        o_ref[...] = _elu(hn + kn)


def _conv2_post(p, cnt, x, root, g2, b2, xs, wlin, gs, bs):
    blk = 1000
    vec = pl.BlockSpec((1, D), lambda ph, i: (0, 0))
    mat = pl.BlockSpec((blk, D), lambda ph, i: (i, 0))
    return pl.pallas_call(
        _conv2_body,
        grid=(2, N // blk),
        in_specs=[pl.BlockSpec((NC, blk, ACC_W), lambda ph, i: (0, i, 0)),
                  pl.BlockSpec((NC, blk, ACC_W), lambda ph, i: (0, i, 0)),
                  mat,
                  pl.BlockSpec((D, D), lambda ph, i: (0, 0)),
                  vec, vec,
                  mat,
                  pl.BlockSpec((D, D), lambda ph, i: (0, 0)),
                  vec, vec],
        out_specs=mat,
        out_shape=jax.ShapeDtypeStruct((N, D), jnp.float32),
        scratch_shapes=[pltpu.VMEM((N, D), jnp.float32),
                        pltpu.VMEM((N, D), jnp.float32),
                        pltpu.VMEM((1, D), jnp.float32),
                        pltpu.VMEM((1, D), jnp.float32),
                        pltpu.VMEM((1, D), jnp.float32),
                        pltpu.VMEM((1, D), jnp.float32)],
    )(p, cnt, x, root, g2.reshape(1, D), b2.reshape(1, D),
      xs, wlin, gs.reshape(1, D), bs.reshape(1, D))


def _post(p, cnt, x, root):
    blk = 1000
    return pl.pallas_call(
        _post_body,
        grid=(N // blk,),
        in_specs=[pl.BlockSpec((NC, blk, ACC_W), lambda i: (0, i, 0)),
                  pl.BlockSpec((NC, blk, ACC_W), lambda i: (0, i, 0)),
                  pl.BlockSpec((blk, D), lambda i: (i, 0)),
                  pl.BlockSpec((D, D), lambda i: (0, 0))],
        out_specs=[pl.BlockSpec((blk, D), lambda i: (i, 0)),
                   pl.BlockSpec((1, D), lambda i: (0, 0)),
                   pl.BlockSpec((1, D), lambda i: (0, 0))],
        out_shape=[jax.ShapeDtypeStruct((N, D), jnp.float32),
                   jax.ShapeDtypeStruct((1, D), jnp.float32),
                   jax.ShapeDtypeStruct((1, D), jnp.float32)],
    )(p, cnt, x, root)


def _mmstats_body(x_ref, w_ref, h_ref, s_ref, q_ref):
    i = pl.program_id(0)
    h = jnp.dot(x_ref[...], w_ref[...], preferred_element_type=jnp.float32)
    h_ref[...] = h

    @pl.when(i == 0)
    def _():
        s_ref[...] = jnp.zeros_like(s_ref)
        q_ref[...] = jnp.zeros_like(q_ref)

    s_ref[...] += jnp.sum(h, axis=0, keepdims=True)
    q_ref[...] += jnp.sum(h * h, axis=0, keepdims=True)


def _mmstats(x, w):
    blk = 1000
    return pl.pallas_call(
        _mmstats_body,
        grid=(N // blk,),
        in_specs=[pl.BlockSpec((blk, D), lambda i: (i, 0)),
                  pl.BlockSpec((D, D), lambda i: (0, 0))],
        out_specs=[pl.BlockSpec((blk, D), lambda i: (i, 0)),
                   pl.BlockSpec((1, D), lambda i: (0, 0)),
                   pl.BlockSpec((1, D), lambda i: (0, 0))],
        out_shape=[jax.ShapeDtypeStruct((N, D), jnp.float32),
                   jax.ShapeDtypeStruct((1, D), jnp.float32),
                   jax.ShapeDtypeStruct((1, D), jnp.float32)],
    )(x, w)


def _norm(h, s, q, g, b):
    mean = s * (1.0 / N)
    var = q * (1.0 / N) - mean * mean
    inv = lax.rsqrt(var + EPS)
    return (h - mean) * inv * g + b


def _elu(y):
    return jnp.where(y > 0, y, jnp.exp(jnp.minimum(y, 0.0)) - 1.0)


def _bn_elu_body(h_ref, s_ref, q_ref, g_ref, b_ref, o_ref):
    o_ref[...] = _elu(_norm(h_ref[...], s_ref[...], q_ref[...],
                            g_ref[...], b_ref[...]))


def _bn_elu(h, s, q, g, b):
    blk = 1000
    vec = pl.BlockSpec((1, D), lambda i: (0, 0))
    return pl.pallas_call(
        _bn_elu_body,
        grid=(N // blk,),
        in_specs=[pl.BlockSpec((blk, D), lambda i: (i, 0)), vec, vec, vec, vec],
        out_specs=pl.BlockSpec((blk, D), lambda i: (i, 0)),
        out_shape=jax.ShapeDtypeStruct((N, D), jnp.float32),
    )(h, s, q, g.reshape(1, D), b.reshape(1, D))


def _bn2_elu_body(h_ref, s2_ref, q2_ref, g2_ref, b2_ref,
                  k_ref, ss_ref, qs_ref, gs_ref, bs_ref, o_ref):
    hn = _norm(h_ref[...], s2_ref[...], q2_ref[...], g2_ref[...], b2_ref[...])
    kn = _norm(k_ref[...], ss_ref[...], qs_ref[...], gs_ref[...], bs_ref[...])
    o_ref[...] = _elu(hn + kn)


def _bn2_elu(h, s2, q2, g2, b2, sk, ss, qs, gs, bs):
    blk = 1000
    mat = pl.BlockSpec((blk, D), lambda i: (i, 0))
    vec = pl.BlockSpec((1, D), lambda i: (0, 0))
    return pl.pallas_call(
        _bn2_elu_body,
        grid=(N // blk,),
        in_specs=[mat, vec, vec, vec, vec, mat, vec, vec, vec, vec],
        out_specs=mat,
        out_shape=jax.ShapeDtypeStruct((N, D), jnp.float32),
    )(h, s2, q2, g2.reshape(1, D), b2.reshape(1, D),
      sk, ss, qs, gs.reshape(1, D), bs.reshape(1, D))


@functools.cache
def _get_sc_agg():
    mesh = plsc.VectorSubcoreMesh(core_axis_name="c", subcore_axis_name="s")

    @functools.partial(
        pl.kernel,
        out_type=(jax.ShapeDtypeStruct((NC, NP, ACC_W), jnp.float32),
                  jax.ShapeDtypeStruct((NC, NP, ACC_W), jnp.float32)),
        mesh=mesh,
        scratch_types=[
            pltpu.VMEM((C,), jnp.int32),
            pltpu.VMEM((C,), jnp.int32),
            pltpu.VMEM((C,), jnp.int32),
            pltpu.VMEM((C,), jnp.int32),
            pltpu.VMEM((C, K), jnp.float32),
            pltpu.VMEM((C, K), jnp.float32),
            pltpu.VMEM((C, KD // 2), jnp.int32),
            pltpu.VMEM((C, KD // 2), jnp.int32),
            pltpu.VMEM((C, ACC_W), jnp.float32),
            pltpu.VMEM((C, ACC_W), jnp.float32),
            pltpu.VMEM((C,), jnp.int32),
            pltpu.VMEM((C,), jnp.int32),
            pltpu.VMEM((ZROWS, ACC_W), jnp.float32),
            pltpu.VMEM((16,), jnp.int32),
            pltpu.VMEM_SHARED((NP, ACC_W), jnp.float32),
            pltpu.SemaphoreType.DMA,
            pltpu.SemaphoreType.DMA,
            pltpu.SemaphoreType.DMA,
            pltpu.SemaphoreType.DMA,
        ],
    )
    def _sc_agg(xn, src, dst, wb, dflag, feat_out, cnt_out,
                srcv0, srcv1, dstv0, dstv1, wv0, wv1, rows0, rows1,
                yv0, yv1, sdst0, sdst1, zbuf, fv, acc,
                gsem0, gsem1, ssem0, ssem1):
        cid = lax.axis_index("c")
        sid = lax.axis_index("s")
        wid = cid * NS + sid

        zero16 = jnp.zeros((16,), jnp.float32)

        def zb(i, carry):
            for j in range(ACC_W // 16):
                zbuf[i, pl.ds(j * 16, 16)] = zero16
            return carry

        lax.fori_loop(0, ZROWS, zb, 0)
        r0 = sid * ROWS_PER_TILE
        for t in range(ROWS_PER_TILE // ZROWS):
            pltpu.sync_copy(zbuf, acc.at[pl.ds(r0 + t * ZROWS, ZROWS)])
        pltpu.sync_copy(dflag.at[pl.ds(0, 16)], fv)
        plsc.subcore_barrier()

        ebase = wid * EPW

        def prefetch(ci, srcv, dstv, wv, rows, gsem):
            base = ebase + ci * C
            pltpu.sync_copy(src.at[pl.ds(base, C)], srcv)
            pltpu.async_copy(dst.at[pl.ds(base, C)], dstv, gsem)
            pltpu.async_copy(wb.at[pl.ds(base, C)], wv, gsem)
            pltpu.async_copy(xn.at[srcv], rows, gsem)

        def gwait(srcv, dstv, wv, rows, gsem):
            pltpu.make_async_copy(dst.at[pl.ds(0, C)], dstv, gsem).wait()
            pltpu.make_async_copy(wb.at[pl.ds(0, C)], wv, gsem).wait()
            pltpu.make_async_copy(xn.at[srcv], rows, gsem).wait()

        def compute(rows, wv, dstv, yv, sdst, ssem):
            def edge(e, ecarry):
                wrow = wv[e, pl.ds(0, K)]
                for g in range(4):
                    alo = zero16
                    ahi = zero16
                    for k in range(K):
                        wk = wrow[k]
                        v = rows[e, pl.ds(k * 64 + g * 16, 16)]
                        lof = lax.bitcast_convert_type(v << 16, jnp.float32)
                        hif = lax.bitcast_convert_type(v, jnp.float32)
                        alo = alo + wk * lof
                        ahi = ahi + wk * hif
                    yv[e, pl.ds(g * 32, 16)] = alo
                    yv[e, pl.ds(g * 32 + 16, 16)] = ahi
                return ecarry

            lax.fori_loop(0, C, edge, 0)
            sdst[pl.ds(0, C)] = dstv[pl.ds(0, C)]
            pltpu.async_copy(yv, acc.at[sdst], ssem, add=True)

        def swait(yv, sdst, ssem):
            pltpu.make_async_copy(yv, acc.at[sdst], ssem).wait()

        prefetch(0, srcv0, dstv0, wv0, rows0, gsem0)
        prefetch(1, srcv1, dstv1, wv1, rows1, gsem1)
        gwait(srcv0, dstv0, wv0, rows0, gsem0)
        compute(rows0, wv0, dstv0, yv0, sdst0, ssem0)
        prefetch(2, srcv0, dstv0, wv0, rows0, gsem0)
        gwait(srcv1, dstv1, wv1, rows1, gsem1)
        compute(rows1, wv1, dstv1, yv1, sdst1, ssem1)

        def pair(p, carry):
            prefetch(2 * p + 1, srcv1, dstv1, wv1, rows1, gsem1)
            gwait(srcv0, dstv0, wv0, rows0, gsem0)
            swait(yv0, sdst0, ssem0)
            compute(rows0, wv0, dstv0, yv0, sdst0, ssem0)
            prefetch(2 * p + 2, srcv0, dstv0, wv0, rows0, gsem0)
            gwait(srcv1, dstv1, wv1, rows1, gsem1)
            swait(yv1, sdst1, ssem1)
            compute(rows1, wv1, dstv1, yv1, sdst1, ssem1)
            return carry

        lax.fori_loop(1, (NCHUNK - 1) // 2, pair, 0)
        gwait(srcv0, dstv0, wv0, rows0, gsem0)
        swait(yv0, sdst0, ssem0)
        compute(rows0, wv0, dstv0, yv0, sdst0, ssem0)
        swait(yv0, sdst0, ssem0)
        swait(yv1, sdst1, ssem1)
        plsc.subcore_barrier()

        pltpu.sync_copy(acc.at[pl.ds(r0, ROWS_PER_TILE)],
                        feat_out.at[cid, pl.ds(r0, ROWS_PER_TILE)])

        flag = fv[pl.ds(0, 16)][0]

        @pl.when(flag == 1)
        def _():
            for t in range(ROWS_PER_TILE // ZROWS):
                pltpu.sync_copy(zbuf, acc.at[pl.ds(r0 + t * ZROWS, ZROWS)])
            plsc.subcore_barrier()
            onevec = jnp.where(lax.iota(jnp.int32, 16) == 0, 1.0, 0.0)

            def ov(i, carry):
                yv0[i, pl.ds(0, 16)] = onevec
                for j in range(1, ACC_W // 16):
                    yv0[i, pl.ds(j * 16, 16)] = zero16
                return carry

            lax.fori_loop(0, C, ov, 0)
            pltpu.async_copy(dst.at[pl.ds(ebase, C)], dstv0, gsem0)

            def cchunk(i, carry):
                pltpu.make_async_copy(dst.at[pl.ds(0, C)], dstv0, gsem0).wait()
                dstv1[pl.ds(0, C)] = dstv0[pl.ds(0, C)]
                pltpu.async_copy(
                    dst.at[pl.ds(ebase + (i + 1) * C, C)], dstv0, gsem0)
                pltpu.sync_copy(yv0, acc.at[dstv1], add=True)
                return carry

            lax.fori_loop(0, NCHUNK - 1, cchunk, 0)
            pltpu.make_async_copy(dst.at[pl.ds(0, C)], dstv0, gsem0).wait()
            pltpu.sync_copy(yv0, acc.at[dstv0], add=True)
            plsc.subcore_barrier()
            pltpu.sync_copy(acc.at[pl.ds(r0, ROWS_PER_TILE)],
                            cnt_out.at[cid, pl.ds(r0, ROWS_PER_TILE)])

    return _sc_agg


def _sc_agg_call(xn, src, dst, wb, dflag):
    return _get_sc_agg()(xn, src, dst, wb, dflag)


def kernel(x, edge_index, edge_attr, W1, root1, g1, b1, W2, root2, g2, b2,
           Wlin, gs, bs):
    src = edge_index[0]
    dst = edge_index[1]
    wb = _basis(edge_attr)
    W13 = jnp.transpose(W1, (1, 0, 2)).reshape(D, K, 4, 32)
    W23 = jnp.transpose(W2, (1, 0, 2)).reshape(D, K, 4, 32)
    W1lo = W13[:, :, :, :16].reshape(D, KD // 2)
    W1hi = W13[:, :, :, 16:].reshape(D, KD // 2)
    W2lo = W23[:, :, :, :16].reshape(D, KD // 2)
    W2hi = W23[:, :, :, 16:].reshape(D, KD // 2)

    flag1 = jnp.ones((16,), jnp.int32)
    flag0 = jnp.zeros((16,), jnp.int32)
    xn1 = _mm_pack(x, W1lo, W1hi)
    p1, cnt1 = _sc_agg_call(xn1, src, dst, wb, flag1)
    h1 = _conv1_post(p1, cnt1, x, root1, g1, b1)

    xn2 = _mm_pack(h1, W2lo, W2hi)
    p2, _ = _sc_agg_call(xn2, src, dst, wb, flag0)
    return _conv2_post(p2, cnt1, h1, root2, g2, b2, x, Wlin, gs, bs)


# speedup vs baseline: 1.0400x; 1.0400x over previous
import functools

import jax
import jax.numpy as jnp
from jax import lax
from jax.experimental import pallas as pl
from jax.experimental.pallas import tpu as pltpu
from jax.experimental.pallas import tpu_sc as plsc

N = 10000
E = 320000
D = 128
K = 16
KD = K * D
ACC_W = 128
NC, NS = 2, 16
NW = NC * NS
EPW = E // NW
C = 16
NCHUNK = EPW // C
NP = 10240
ROWS_PER_TILE = NP // NS
ZROWS = 32
EPS = 1e-5


def _mm_pack_body(x_ref, wlo_ref, whi_ref, o_ref):
    lo = jnp.dot(x_ref[...], wlo_ref[...], preferred_element_type=jnp.float32)
    hi = jnp.dot(x_ref[...], whi_ref[...], preferred_element_type=jnp.float32)
    lou = lax.bitcast_convert_type(lo.astype(jnp.bfloat16),
                                   jnp.uint16).astype(jnp.uint32)
    hiu = lax.bitcast_convert_type(hi.astype(jnp.bfloat16),
                                   jnp.uint16).astype(jnp.uint32)
    o_ref[...] = lax.bitcast_convert_type(lou | (hiu << 16), jnp.int32)


def _mm_pack(x, wlo, whi):
    n, d = x.shape
    p = wlo.shape[1]
    blk = 1000
    return pl.pallas_call(
        _mm_pack_body,
        grid=(n // blk,),
        in_specs=[pl.BlockSpec((blk, d), lambda i: (i, 0)),
                  pl.BlockSpec((d, p), lambda i: (0, 0)),
                  pl.BlockSpec((d, p), lambda i: (0, 0))],
        out_specs=pl.BlockSpec((blk, p), lambda i: (i, 0)),
        out_shape=jax.ShapeDtypeStruct((n, p), jnp.int32),
    )(x, wlo, whi)


def _basis_body(a_ref, o_ref):
    a = a_ref[...]
    kk = lax.broadcasted_iota(jnp.int32, (a.shape[0], K), 1)
    w = jnp.ones((a.shape[0], K), jnp.float32)
    for d in range(4):
        ad = a[:, d:d + 1]
        w = w * jnp.where((kk >> d) & 1 == 1, ad, 1.0 - ad)
    o_ref[...] = w


def _basis(edge_attr):
    blk = 4000
    return pl.pallas_call(
        _basis_body,
        grid=(E // blk,),
        in_specs=[pl.BlockSpec((blk, 4), lambda i: (i, 0))],
        out_specs=pl.BlockSpec((blk, K), lambda i: (i, 0)),
        out_shape=jax.ShapeDtypeStruct((E, K), jnp.float32),
    )(edge_attr)


def _conv1_body(p_ref, c_ref, x_ref, r_ref, g_ref, b_ref, o_ref,
                hbuf, s_ref, q_ref):
    ph = pl.program_id(0)
    i = pl.program_id(1)

    @pl.when(ph == 0)
    def _():
        p = p_ref[0] + p_ref[1]
        cnt = c_ref[0, :, 0:1] + c_ref[1, :, 0:1]
        agg = p / jnp.maximum(cnt, 1.0)
        h = agg + jnp.dot(x_ref[...], r_ref[...],
                          preferred_element_type=jnp.float32)
        hbuf[pl.ds(i * 1000, 1000), :] = h

        @pl.when(i == 0)
        def _():
            s_ref[...] = jnp.zeros_like(s_ref)
            q_ref[...] = jnp.zeros_like(q_ref)

        s_ref[...] += jnp.sum(h, axis=0, keepdims=True)
        q_ref[...] += jnp.sum(h * h, axis=0, keepdims=True)

    @pl.when(ph == 1)
    def _():
        h = hbuf[pl.ds(i * 1000, 1000), :]
        o_ref[...] = _elu(_norm(h, s_ref[...], q_ref[...],
                                g_ref[...], b_ref[...]))


def _conv1_post(p, cnt, x, root, g, b):
    blk = 1000
    vec = pl.BlockSpec((1, D), lambda ph, i: (0, 0))
    return pl.pallas_call(
        _conv1_body,
        grid=(2, N // blk),
        in_specs=[pl.BlockSpec((NC, blk, ACC_W), lambda ph, i: (0, i, 0)),
                  pl.BlockSpec((NC, blk, ACC_W), lambda ph, i: (0, i, 0)),
                  pl.BlockSpec((blk, D), lambda ph, i: (i, 0)),
                  pl.BlockSpec((D, D), lambda ph, i: (0, 0)),
                  vec, vec],
        out_specs=pl.BlockSpec((blk, D), lambda ph, i: (i, 0)),
        out_shape=jax.ShapeDtypeStruct((N, D), jnp.float32),
        scratch_shapes=[pltpu.VMEM((N, D), jnp.float32),
                        pltpu.VMEM((1, D), jnp.float32),
                        pltpu.VMEM((1, D), jnp.float32)],
    )(p, cnt, x, root, g.reshape(1, D), b.reshape(1, D))


def _conv2_body(p_ref, c_ref, x_ref, r_ref, g2_ref, b2_ref,
                xs_ref, wl_ref, gs_ref, bs_ref, o_ref,
                hbuf, kbuf, s_ref, q_ref, ss_ref, qs_ref):
    ph = pl.program_id(0)
    i = pl.program_id(1)

    @pl.when(ph == 0)
    def _():
        p = p_ref[0] + p_ref[1]
        cnt = c_ref[0, :, 0:1] + c_ref[1, :, 0:1]
        agg = p / jnp.maximum(cnt, 1.0)
        h = agg + jnp.dot(x_ref[...], r_ref[...],
                          preferred_element_type=jnp.float32)
        hbuf[pl.ds(i * 1000, 1000), :] = h
        sk = jnp.dot(xs_ref[...], wl_ref[...],
                     preferred_element_type=jnp.float32)
        kbuf[pl.ds(i * 1000, 1000), :] = sk

        @pl.when(i == 0)
        def _():
            s_ref[...] = jnp.zeros_like(s_ref)
            q_ref[...] = jnp.zeros_like(q_ref)
            ss_ref[...] = jnp.zeros_like(ss_ref)
            qs_ref[...] = jnp.zeros_like(qs_ref)

        s_ref[...] += jnp.sum(h, axis=0, keepdims=True)
        q_ref[...] += jnp.sum(h * h, axis=0, keepdims=True)
        ss_ref[...] += jnp.sum(sk, axis=0, keepdims=True)
        qs_ref[...] += jnp.sum(sk * sk, axis=0, keepdims=True)

    @pl.when(ph == 1)
    def _():
        h = hbuf[pl.ds(i * 1000, 1000), :]
        sk = kbuf[pl.ds(i * 1000, 1000), :]
        hn = _norm(h, s_ref[...], q_ref[...], g2_ref[...], b2_ref[...])
        kn = _norm(sk, ss_ref[...], qs_ref[...], gs_ref[...], bs_ref[...])
        o_ref[...] = _elu(hn + kn)


def _conv2_post(p, cnt, x, root, g2, b2, xs, wlin, gs, bs):
    blk = 1000
    vec = pl.BlockSpec((1, D), lambda ph, i: (0, 0))
    mat = pl.BlockSpec((blk, D), lambda ph, i: (i, 0))
    return pl.pallas_call(
        _conv2_body,
        grid=(2, N // blk),
        in_specs=[pl.BlockSpec((NC, blk, ACC_W), lambda ph, i: (0, i, 0)),
                  pl.BlockSpec((NC, blk, ACC_W), lambda ph, i: (0, i, 0)),
                  mat,
                  pl.BlockSpec((D, D), lambda ph, i: (0, 0)),
                  vec, vec,
                  mat,
                  pl.BlockSpec((D, D), lambda ph, i: (0, 0)),
                  vec, vec],
        out_specs=mat,
        out_shape=jax.ShapeDtypeStruct((N, D), jnp.float32),
        scratch_shapes=[pltpu.VMEM((N, D), jnp.float32),
                        pltpu.VMEM((N, D), jnp.float32),
                        pltpu.VMEM((1, D), jnp.float32),
                        pltpu.VMEM((1, D), jnp.float32),
                        pltpu.VMEM((1, D), jnp.float32),
                        pltpu.VMEM((1, D), jnp.float32)],
    )(p, cnt, x, root, g2.reshape(1, D), b2.reshape(1, D),
      xs, wlin, gs.reshape(1, D), bs.reshape(1, D))


def _post(p, cnt, x, root):
    blk = 1000
    return pl.pallas_call(
        _post_body,
        grid=(N // blk,),
        in_specs=[pl.BlockSpec((NC, blk, ACC_W), lambda i: (0, i, 0)),
                  pl.BlockSpec((NC, blk, ACC_W), lambda i: (0, i, 0)),
                  pl.BlockSpec((blk, D), lambda i: (i, 0)),
                  pl.BlockSpec((D, D), lambda i: (0, 0))],
        out_specs=[pl.BlockSpec((blk, D), lambda i: (i, 0)),
                   pl.BlockSpec((1, D), lambda i: (0, 0)),
                   pl.BlockSpec((1, D), lambda i: (0, 0))],
        out_shape=[jax.ShapeDtypeStruct((N, D), jnp.float32),
                   jax.ShapeDtypeStruct((1, D), jnp.float32),
                   jax.ShapeDtypeStruct((1, D), jnp.float32)],
    )(p, cnt, x, root)


def _mmstats_body(x_ref, w_ref, h_ref, s_ref, q_ref):
    i = pl.program_id(0)
    h = jnp.dot(x_ref[...], w_ref[...], preferred_element_type=jnp.float32)
    h_ref[...] = h

    @pl.when(i == 0)
    def _():
        s_ref[...] = jnp.zeros_like(s_ref)
        q_ref[...] = jnp.zeros_like(q_ref)

    s_ref[...] += jnp.sum(h, axis=0, keepdims=True)
    q_ref[...] += jnp.sum(h * h, axis=0, keepdims=True)


def _mmstats(x, w):
    blk = 1000
    return pl.pallas_call(
        _mmstats_body,
        grid=(N // blk,),
        in_specs=[pl.BlockSpec((blk, D), lambda i: (i, 0)),
                  pl.BlockSpec((D, D), lambda i: (0, 0))],
        out_specs=[pl.BlockSpec((blk, D), lambda i: (i, 0)),
                   pl.BlockSpec((1, D), lambda i: (0, 0)),
                   pl.BlockSpec((1, D), lambda i: (0, 0))],
        out_shape=[jax.ShapeDtypeStruct((N, D), jnp.float32),
                   jax.ShapeDtypeStruct((1, D), jnp.float32),
                   jax.ShapeDtypeStruct((1, D), jnp.float32)],
    )(x, w)


def _norm(h, s, q, g, b):
    mean = s * (1.0 / N)
    var = q * (1.0 / N) - mean * mean
    inv = lax.rsqrt(var + EPS)
    return (h - mean) * inv * g + b


def _elu(y):
    return jnp.where(y > 0, y, jnp.exp(jnp.minimum(y, 0.0)) - 1.0)


def _bn_elu_body(h_ref, s_ref, q_ref, g_ref, b_ref, o_ref):
    o_ref[...] = _elu(_norm(h_ref[...], s_ref[...], q_ref[...],
                            g_ref[...], b_ref[...]))


def _bn_elu(h, s, q, g, b):
    blk = 1000
    vec = pl.BlockSpec((1, D), lambda i: (0, 0))
    return pl.pallas_call(
        _bn_elu_body,
        grid=(N // blk,),
        in_specs=[pl.BlockSpec((blk, D), lambda i: (i, 0)), vec, vec, vec, vec],
        out_specs=pl.BlockSpec((blk, D), lambda i: (i, 0)),
        out_shape=jax.ShapeDtypeStruct((N, D), jnp.float32),
    )(h, s, q, g.reshape(1, D), b.reshape(1, D))


def _bn2_elu_body(h_ref, s2_ref, q2_ref, g2_ref, b2_ref,
                  k_ref, ss_ref, qs_ref, gs_ref, bs_ref, o_ref):
    hn = _norm(h_ref[...], s2_ref[...], q2_ref[...], g2_ref[...], b2_ref[...])
    kn = _norm(k_ref[...], ss_ref[...], qs_ref[...], gs_ref[...], bs_ref[...])
    o_ref[...] = _elu(hn + kn)


def _bn2_elu(h, s2, q2, g2, b2, sk, ss, qs, gs, bs):
    blk = 1000
    mat = pl.BlockSpec((blk, D), lambda i: (i, 0))
    vec = pl.BlockSpec((1, D), lambda i: (0, 0))
    return pl.pallas_call(
        _bn2_elu_body,
        grid=(N // blk,),
        in_specs=[mat, vec, vec, vec, vec, mat, vec, vec, vec, vec],
        out_specs=mat,
        out_shape=jax.ShapeDtypeStruct((N, D), jnp.float32),
    )(h, s2, q2, g2.reshape(1, D), b2.reshape(1, D),
      sk, ss, qs, gs.reshape(1, D), bs.reshape(1, D))


@functools.cache
def _get_sc_agg():
    mesh = plsc.VectorSubcoreMesh(core_axis_name="c", subcore_axis_name="s")

    @functools.partial(
        pl.kernel,
        out_type=(jax.ShapeDtypeStruct((NC, NP, ACC_W), jnp.float32),
                  jax.ShapeDtypeStruct((NC, NP, ACC_W), jnp.float32)),
        mesh=mesh,
        scratch_types=[
            pltpu.VMEM((C,), jnp.int32),
            pltpu.VMEM((C,), jnp.int32),
            pltpu.VMEM((C,), jnp.int32),
            pltpu.VMEM((C,), jnp.int32),
            pltpu.VMEM((C, K), jnp.float32),
            pltpu.VMEM((C, K), jnp.float32),
            pltpu.VMEM((C, KD // 2), jnp.int32),
            pltpu.VMEM((C, KD // 2), jnp.int32),
            pltpu.VMEM((C, ACC_W), jnp.float32),
            pltpu.VMEM((C, ACC_W), jnp.float32),
            pltpu.VMEM((C,), jnp.int32),
            pltpu.VMEM((C,), jnp.int32),
            pltpu.VMEM((ZROWS, ACC_W), jnp.float32),
            pltpu.VMEM((16,), jnp.int32),
            pltpu.VMEM_SHARED((NP, ACC_W), jnp.float32),
            pltpu.SemaphoreType.DMA,
            pltpu.SemaphoreType.DMA,
            pltpu.SemaphoreType.DMA,
            pltpu.SemaphoreType.DMA,
        ],
    )
    def _sc_agg(xn, src, dst, wb, dflag, feat_out, cnt_out,
                srcv0, srcv1, dstv0, dstv1, wv0, wv1, rows0, rows1,
                yv0, yv1, sdst0, sdst1, zbuf, fv, acc,
                gsem0, gsem1, ssem0, ssem1):
        cid = lax.axis_index("c")
        sid = lax.axis_index("s")
        wid = cid * NS + sid

        zero16 = jnp.zeros((16,), jnp.float32)

        def zb(i, carry):
            for j in range(ACC_W // 16):
                zbuf[i, pl.ds(j * 16, 16)] = zero16
            return carry

        lax.fori_loop(0, ZROWS, zb, 0)
        r0 = sid * ROWS_PER_TILE
        for t in range(ROWS_PER_TILE // ZROWS):
            pltpu.sync_copy(zbuf, acc.at[pl.ds(r0 + t * ZROWS, ZROWS)])
        pltpu.sync_copy(dflag.at[pl.ds(0, 16)], fv)
        plsc.subcore_barrier()

        ebase = wid * EPW

        def prefetch(ci, srcv, dstv, wv, rows, gsem):
            base = ebase + ci * C
            pltpu.sync_copy(src.at[pl.ds(base, C)], srcv)
            pltpu.async_copy(dst.at[pl.ds(base, C)], dstv, gsem)
            pltpu.async_copy(wb.at[pl.ds(base, C)], wv, gsem)
            pltpu.async_copy(xn.at[srcv], rows, gsem)

        def gwait(srcv, dstv, wv, rows, gsem):
            pltpu.make_async_copy(dst.at[pl.ds(0, C)], dstv, gsem).wait()
            pltpu.make_async_copy(wb.at[pl.ds(0, C)], wv, gsem).wait()
            pltpu.make_async_copy(xn.at[srcv], rows, gsem).wait()

        def compute(rows, wv, dstv, yv, sdst, ssem):
            def edge(e, ecarry):
                wrow = wv[e, pl.ds(0, K)]
                for g in range(4):
                    alo = zero16
                    ahi = zero16
                    for k in range(K):
                        wk = wrow[k]
                        v = rows[e, pl.ds(k * 64 + g * 16, 16)]
                        lof = lax.bitcast_convert_type(v << 16, jnp.float32)
                        hif = lax.bitcast_convert_type(v, jnp.float32)
                        alo = alo + wk * lof
                        ahi = ahi + wk * hif
                    yv[e, pl.ds(g * 32, 16)] = alo
                    yv[e, pl.ds(g * 32 + 16, 16)] = ahi
                return ecarry

            lax.fori_loop(0, C, edge, 0)
            sdst[pl.ds(0, C)] = dstv[pl.ds(0, C)]
            pltpu.async_copy(yv, acc.at[sdst], ssem, add=True)

        def swait(yv, sdst, ssem):
            pltpu.make_async_copy(yv, acc.at[sdst], ssem).wait()

        prefetch(0, srcv0, dstv0, wv0, rows0, gsem0)
        prefetch(1, srcv1, dstv1, wv1, rows1, gsem1)
        gwait(srcv0, dstv0, wv0, rows0, gsem0)
        compute(rows0, wv0, dstv0, yv0, sdst0, ssem0)
        prefetch(2, srcv0, dstv0, wv0, rows0, gsem0)
        gwait(srcv1, dstv1, wv1, rows1, gsem1)
        compute(rows1, wv1, dstv1, yv1, sdst1, ssem1)

        def pair(p, carry):
            prefetch(2 * p + 1, srcv1, dstv1, wv1, rows1, gsem1)
            gwait(srcv0, dstv0, wv0, rows0, gsem0)
            swait(yv0, sdst0, ssem0)
            compute(rows0, wv0, dstv0, yv0, sdst0, ssem0)
            prefetch(2 * p + 2, srcv0, dstv0, wv0, rows0, gsem0)
            gwait(srcv1, dstv1, wv1, rows1, gsem1)
            swait(yv1, sdst1, ssem1)
            compute(rows1, wv1, dstv1, yv1, sdst1, ssem1)
            return carry

        lax.fori_loop(1, (NCHUNK - 1) // 2, pair, 0)
        gwait(srcv0, dstv0, wv0, rows0, gsem0)
        swait(yv0, sdst0, ssem0)
        compute(rows0, wv0, dstv0, yv0, sdst0, ssem0)
        swait(yv0, sdst0, ssem0)
        swait(yv1, sdst1, ssem1)
        plsc.subcore_barrier()

        pltpu.sync_copy(acc.at[pl.ds(r0, ROWS_PER_TILE)],
                        feat_out.at[cid, pl.ds(r0, ROWS_PER_TILE)])

        flag = fv[pl.ds(0, 16)][0]

        @pl.when(flag == 1)
        def _():
            for t in range(ROWS_PER_TILE // ZROWS):
                pltpu.sync_copy(zbuf, acc.at[pl.ds(r0 + t * ZROWS, ZROWS)])
            plsc.subcore_barrier()
            onevec = jnp.where(lax.iota(jnp.int32, 16) == 0, 1.0, 0.0)

            def ov(i, carry):
                yv0[i, pl.ds(0, 16)] = onevec
                yv1[i, pl.ds(0, 16)] = onevec
                for j in range(1, ACC_W // 16):
                    yv0[i, pl.ds(j * 16, 16)] = zero16
                    yv1[i, pl.ds(j * 16, 16)] = zero16
                return carry

            lax.fori_loop(0, C, ov, 0)

            def cpre(ci, dstv, gsem):
                pltpu.async_copy(dst.at[pl.ds(ebase + ci * C, C)], dstv, gsem)

            def cwait(dstv, gsem):
                pltpu.make_async_copy(dst.at[pl.ds(0, C)], dstv, gsem).wait()

            cpre(0, dstv0, gsem0)
            cpre(1, dstv1, gsem1)
            cwait(dstv0, gsem0)
            sdst0[pl.ds(0, C)] = dstv0[pl.ds(0, C)]
            cpre(2, dstv0, gsem0)
            pltpu.async_copy(yv0, acc.at[sdst0], ssem0, add=True)
            cwait(dstv1, gsem1)
            sdst1[pl.ds(0, C)] = dstv1[pl.ds(0, C)]
            cpre(3, dstv1, gsem1)
            pltpu.async_copy(yv1, acc.at[sdst1], ssem1, add=True)

            def cpair(p, carry):
                cwait(dstv0, gsem0)
                swait(yv0, sdst0, ssem0)
                sdst0[pl.ds(0, C)] = dstv0[pl.ds(0, C)]
                cpre(2 * p + 2, dstv0, gsem0)
                pltpu.async_copy(yv0, acc.at[sdst0], ssem0, add=True)
                cwait(dstv1, gsem1)
                swait(yv1, sdst1, ssem1)
                sdst1[pl.ds(0, C)] = dstv1[pl.ds(0, C)]
                cpre(2 * p + 3, dstv1, gsem1)
                pltpu.async_copy(yv1, acc.at[sdst1], ssem1, add=True)
                return carry

            lax.fori_loop(1, (NCHUNK - 3) // 2, cpair, 0)
            cwait(dstv0, gsem0)
            swait(yv0, sdst0, ssem0)
            sdst0[pl.ds(0, C)] = dstv0[pl.ds(0, C)]
            cpre(NCHUNK - 1, dstv0, gsem0)
            pltpu.async_copy(yv0, acc.at[sdst0], ssem0, add=True)
            cwait(dstv1, gsem1)
            swait(yv1, sdst1, ssem1)
            sdst1[pl.ds(0, C)] = dstv1[pl.ds(0, C)]
            pltpu.async_copy(yv1, acc.at[sdst1], ssem1, add=True)
            cwait(dstv0, gsem0)
            swait(yv0, sdst0, ssem0)
            sdst0[pl.ds(0, C)] = dstv0[pl.ds(0, C)]
            pltpu.async_copy(yv0, acc.at[sdst0], ssem0, add=True)
            swait(yv0, sdst0, ssem0)
            swait(yv1, sdst1, ssem1)
            plsc.subcore_barrier()
            pltpu.sync_copy(acc.at[pl.ds(r0, ROWS_PER_TILE)],
                            cnt_out.at[cid, pl.ds(r0, ROWS_PER_TILE)])

    return _sc_agg


def _sc_agg_call(xn, src, dst, wb, dflag):
    return _get_sc_agg()(xn, src, dst, wb, dflag)


def kernel(x, edge_index, edge_attr, W1, root1, g1, b1, W2, root2, g2, b2,
           Wlin, gs, bs):
    src = edge_index[0]
    dst = edge_index[1]
    wb = _basis(edge_attr)
    W13 = jnp.transpose(W1, (1, 0, 2)).reshape(D, K, 4, 32)
    W23 = jnp.transpose(W2, (1, 0, 2)).reshape(D, K, 4, 32)
    W1lo = W13[:, :, :, :16].reshape(D, KD // 2)
    W1hi = W13[:, :, :, 16:].reshape(D, KD // 2)
    W2lo = W23[:, :, :, :16].reshape(D, KD // 2)
    W2hi = W23[:, :, :, 16:].reshape(D, KD // 2)

    flag1 = jnp.ones((16,), jnp.int32)
    flag0 = jnp.zeros((16,), jnp.int32)
    xn1 = _mm_pack(x, W1lo, W1hi)
    p1, cnt1 = _sc_agg_call(xn1, src, dst, wb, flag1)
    h1 = _conv1_post(p1, cnt1, x, root1, g1, b1)

    xn2 = _mm_pack(h1, W2lo, W2hi)
    p2, _ = _sc_agg_call(xn2, src, dst, wb, flag0)
    return _conv2_post(p2, cnt1, h1, root2, g2, b2, x, Wlin, gs, bs)
